# baseline (speedup 1.0000x reference)
import functools

import jax
import jax.numpy as jnp
from jax import lax
from jax.experimental import pallas as pl
from jax.experimental.pallas import tpu as pltpu
from jax.experimental.pallas import tpu_sc as plsc

B = 16384
S = 77
D = 64
DP = 128
EOS = 49407
NC = 2
NS = 16
NW = NC * NS
BBLK = 128
NBLK = B // BBLK
BLK_PER_W = NBLK // NW
LANES = 16


def _emb_body(ids_hbm, table_hbm, pos_hbm, out_hbm, pos_v, idx_v,
              rv0, rv1, sb0, sb1, sg0, sg1, ss0, ss1):
    rv = [rv0, rv1]
    sb = [sb0, sb1]
    sg = [sg0, sg1]
    ss = [ss0, ss1]

    wid = lax.axis_index("s") * NC + lax.axis_index("c")

    pltpu.sync_copy(pos_hbm, pos_v)

    iota = lax.iota(jnp.int32, LANES)

    def start_gather(p, s):
        pltpu.async_copy(table_hbm.at[idx_v.at[s]], rv[p], sg[p])

    def wait_gather(p, s):
        pltpu.make_async_copy(table_hbm.at[idx_v.at[s]], rv[p], sg[p]).wait()

    def start_stores(c0, p, s):
        for r in range(D // 8):
            pltpu.async_copy(
                sb[p].at[r],
                out_hbm.at[s, pl.ds(r * 8, 8), pl.ds(c0, BBLK)], ss[p])

    def wait_stores(c0, p, s):
        for r in range(D // 8):
            pltpu.make_async_copy(
                sb[p].at[r],
                out_hbm.at[s, pl.ds(r * 8, 8), pl.ds(c0, BBLK)],
                ss[p]).wait()

    def transpose_add(p, s):
        s_vec = jnp.full((LANES,), s, jnp.int32)
        for r in range(D // 8):
            for dd in range(8):
                d = r * 8 + dd
                d_vec = jnp.full((LANES,), d, jnp.int32)
                pv = plsc.load_gather(pos_v, [s_vec, d_vec])
                for k in range(BBLK // LANES):
                    b_vec = iota + (k * LANES)
                    v = plsc.load_gather(rv[p], [b_vec, d_vec])
                    sb[p][r, dd, pl.ds(k * LANES, LANES)] = v + pv

    def do_block(cb, carry):
        c0 = (wid * BLK_PER_W + cb) * BBLK
        pltpu.sync_copy(ids_hbm.at[:, pl.ds(c0, BBLK)], idx_v)
        start_gather(0, 0)
        start_gather(1, 1)

        def s_pair(sp, carry2):
            for par in range(2):
                s = sp * 2 + par
                wait_gather(par, s)

                @pl.when(s >= 2)
                def _():
                    wait_stores(c0, par, s - 2)

                transpose_add(par, s)
                start_stores(c0, par, s)

                @pl.when(s + 2 < S)
                def _():
                    start_gather(par, s + 2)

            return carry2

        lax.fori_loop(0, (S - 1) // 2, s_pair, 0)
        s = S - 1
        wait_gather(0, s)
        wait_stores(c0, 0, s - 2)
        transpose_add(0, s)
        start_stores(c0, 0, s)
        wait_stores(c0, 1, S - 2)
        wait_stores(c0, 0, S - 1)
        return carry

    lax.fori_loop(0, BLK_PER_W, do_block, 0)


_emb = functools.partial(
    pl.kernel,
    out_type=jax.ShapeDtypeStruct((S, D, B), jnp.float32),
    mesh=plsc.VectorSubcoreMesh(core_axis_name="c", subcore_axis_name="s"),
    scratch_types=[
        pltpu.VMEM((S, D), jnp.float32),
        pltpu.VMEM((S, BBLK), jnp.int32),
        pltpu.VMEM((BBLK, DP), jnp.float32),
        pltpu.VMEM((BBLK, DP), jnp.float32),
        pltpu.VMEM((D // 8, 8, BBLK), jnp.float32),
        pltpu.VMEM((D // 8, 8, BBLK), jnp.float32),
        pltpu.SemaphoreType.DMA,
        pltpu.SemaphoreType.DMA,
        pltpu.SemaphoreType.DMA,
        pltpu.SemaphoreType.DMA,
    ],
    compiler_params=pltpu.CompilerParams(needs_layout_passes=False),
)(_emb_body)


def _mask_body(ids_ref, m_ref):
    m_ref[...] = ids_ref[...] == EOS


_mask = pl.pallas_call(
    _mask_body,
    out_shape=jax.ShapeDtypeStruct((B, S), jnp.bool_),
)


def kernel(input_ids, text_embedding, positional_embedding):
    ids = input_ids.astype(jnp.int32)
    ids_t = ids.T
    table_p = jnp.pad(text_embedding, ((0, 0), (0, DP - D)))
    x = _emb(ids_t, table_p, positional_embedding)
    tokens = x.transpose(2, 0, 1)
    mask = _mask(ids)
    return tokens, mask


# speedup vs baseline: 2.2541x; 2.2541x over previous
import functools

import jax
import jax.numpy as jnp
from jax import lax
from jax.experimental import pallas as pl
from jax.experimental.pallas import tpu as pltpu
from jax.experimental.pallas import tpu_sc as plsc

B = 16384
S = 77
D = 64
DP = 128
EOS = 49407
NC = 2
NS = 16
NW = NC * NS
ROWS_PER_W = B // NW
CB = 1
NCHUNK = ROWS_PER_W // CB
NBUF = 4
LANES = 16


def _emb_body(ids_hbm, table_hbm, pos_hbm, out_hbm, pos_v,
              idx0, idx1, idx2, idx3, gb0, gb1, gb2, gb3,
              sb0, sb1, sb2, sb3,
              si0, si1, si2, si3, sg0, sg1, sg2, sg3, ss0, ss1, ss2, ss3):
    idx = [idx0, idx1, idx2, idx3]
    gb = [gb0, gb1, gb2, gb3]
    sb = [sb0, sb1, sb2, sb3]
    si = [si0, si1, si2, si3]
    sg = [sg0, sg1, sg2, sg3]
    ss = [ss0, ss1, ss2, ss3]

    wid = lax.axis_index("s") * NC + lax.axis_index("c")
    base_row = wid * ROWS_PER_W

    pltpu.sync_copy(pos_hbm, pos_v)

    def start_idx(b, g):
        row0 = base_row + g * CB
        pltpu.async_copy(ids_hbm.at[pl.ds(row0, CB)], idx[b], si[b])

    def wait_idx(b, g):
        row0 = base_row + g * CB
        pltpu.make_async_copy(ids_hbm.at[pl.ds(row0, CB)], idx[b],
                              si[b]).wait()

    def start_gathers(b):
        for c in range(CB):
            pltpu.async_copy(table_hbm.at[idx[b].at[c]], gb[b].at[c], sg[b])

    def wait_gathers(b):
        for c in range(CB):
            pltpu.make_async_copy(table_hbm.at[idx[b].at[c]], gb[b].at[c],
                                  sg[b]).wait()

    def add_pos(b):
        def s_body(s, carry):
            for c in range(CB):
                for j in range(D // LANES):
                    p = pos_v[s, pl.ds(j * LANES, LANES)]
                    sb[b][c, s, pl.ds(j * LANES, LANES)] = (
                        gb[b][c, s, pl.ds(j * LANES, LANES)] + p)
            return carry

        lax.fori_loop(0, S, s_body, 0)

    def start_store(b, g):
        row0 = base_row + g * CB
        pltpu.async_copy(sb[b], out_hbm.at[pl.ds(row0, CB)], ss[b])

    def wait_store(b, g):
        row0 = base_row + g * CB
        pltpu.make_async_copy(sb[b], out_hbm.at[pl.ds(row0, CB)], ss[b]).wait()

    for h in range(3):
        start_idx(h, h)
    for h in range(2):
        wait_idx(h, h)
        start_gathers(h)

    def outer_body(i, carry):
        for bb in range(NBUF):
            g = i * NBUF + bb
            b = bb
            wait_gathers(b)
            add_pos(b)
            start_store(b, g)
            bn = (bb + 2) % NBUF
            bi = (bb + 3) % NBUF

            @pl.when(g + 2 < NCHUNK)
            def _():
                @pl.when(g >= 2)
                def _():
                    wait_store(bn, g - 2)

                wait_idx(bn, g + 2)
                start_gathers(bn)

            @pl.when(g + 3 < NCHUNK)
            def _():
                start_idx(bi, g + 3)

        return carry

    lax.fori_loop(0, NCHUNK // NBUF, outer_body, 0)

    for k in range(NBUF):
        g = NCHUNK - NBUF + k
        wait_store(g % NBUF, g)


_scr_idx = [pltpu.VMEM((CB, S), jnp.int32) for _ in range(NBUF)]
_scr_gb = [pltpu.VMEM((CB, S, DP), jnp.float32) for _ in range(NBUF)]
_scr_sb = [pltpu.VMEM((CB, S, D), jnp.float32) for _ in range(NBUF)]
_scr_sem = [pltpu.SemaphoreType.DMA for _ in range(3 * NBUF)]

_emb = functools.partial(
    pl.kernel,
    out_type=jax.ShapeDtypeStruct((B, S, D), jnp.float32),
    mesh=plsc.VectorSubcoreMesh(core_axis_name="c", subcore_axis_name="s"),
    scratch_types=[pltpu.VMEM((S, D), jnp.float32)]
    + _scr_idx + _scr_gb + _scr_sb + _scr_sem,
)(_emb_body)


def _mask_body(ids_ref, m_ref):
    m_ref[...] = ids_ref[...] == EOS


_mask = pl.pallas_call(
    _mask_body,
    out_shape=jax.ShapeDtypeStruct((B, S), jnp.bool_),
)


def kernel(input_ids, text_embedding, positional_embedding):
    ids = input_ids.astype(jnp.int32)
    table_p = jnp.pad(text_embedding, ((0, 0), (0, DP - D)))
    tokens = _emb(ids, table_p, positional_embedding)
    mask = _mask(ids)
    return tokens, mask
